# baseline (speedup 1.0000x reference)
import functools
import math

import jax
import jax.numpy as jnp
from jax.experimental import pallas as pl
from jax.experimental.pallas import tpu as pltpu


def _router_body(capacity, xt_ref, cw_ref, mask_ref, w_ref, r_ref):
    i = pl.program_id(0)
    E, T = xt_ref.shape
    EB = cw_ref.shape[0]

    @pl.when(i == 0)
    def _compute_metadata():
        x = xt_ref[...]
        m = jnp.max(x, axis=0, keepdims=True)
        ex = jnp.exp(x - m)
        p = ex / jnp.sum(ex, axis=0, keepdims=True)

        eids = jax.lax.broadcasted_iota(jnp.int32, (E, T), 0)
        p1 = jnp.max(p, axis=0, keepdims=True)
        idx1 = jnp.min(jnp.where(p == p1, eids, E), axis=0, keepdims=True)
        m1 = eids == idx1
        pm = jnp.where(m1, -jnp.inf, p)
        p2 = jnp.max(pm, axis=0, keepdims=True)
        idx2 = jnp.min(jnp.where(pm == p2, eids, E), axis=0, keepdims=True)
        m2 = eids == idx2

        def _cumsum1(v):
            s = 1
            while s < v.shape[1]:
                z = jnp.zeros((v.shape[0], s), v.dtype)
                v = v + jnp.concatenate([z, v[:, :-s]], axis=1)
                s *= 2
            return v

        c1 = _cumsum1(m1.astype(jnp.int32))
        c2 = _cumsum1(m2.astype(jnp.int32))
        total1 = c1[:, T - 1:T]
        rank1 = jnp.sum(jnp.where(m1, c1, 0), axis=0, keepdims=True) - 1
        rank2 = jnp.sum(jnp.where(m2, c2 + total1, 0), axis=0, keepdims=True) - 1

        keep1 = m1 & (rank1 < capacity)
        keep2 = m2 & (rank2 < capacity)
        w_ref[...] = jnp.where(keep1 | keep2, p, 0.0)
        r_ref[...] = (jnp.where(keep1, rank1, -1)
                      + jnp.where(keep2, rank2 + 1, 0))

    w = w_ref[pl.ds(i * EB, EB), :][:, None, :]
    r = r_ref[pl.ds(i * EB, EB), :][:, None, :]
    jc = jax.lax.broadcasted_iota(jnp.int32, (EB, capacity, T), 1)
    out = jnp.where(jc == r, w, 0.0)
    cw_ref[...] = out
    mask_ref[...] = out != 0.0


@jax.jit
def kernel(inputs):
    T, E = inputs.shape
    capacity = math.floor(2.0 * T / E)
    capacity += capacity % 2
    capacity = max(capacity, 4)

    xt = jnp.swapaxes(inputs.astype(jnp.float32), 0, 1)
    EB = 4
    cw_ect, mask_ect = pl.pallas_call(
        functools.partial(_router_body, capacity),
        grid=(E // EB,),
        in_specs=[pl.BlockSpec((E, T), lambda i: (0, 0))],
        out_specs=[
            pl.BlockSpec((EB, capacity, T), lambda i: (i, 0, 0)),
            pl.BlockSpec((EB, capacity, T), lambda i: (i, 0, 0)),
        ],
        out_shape=[
            jax.ShapeDtypeStruct((E, capacity, T), jnp.float32),
            jax.ShapeDtypeStruct((E, capacity, T), jnp.bool_),
        ],
        scratch_shapes=[
            pltpu.VMEM((E, T), jnp.float32),
            pltpu.VMEM((E, T), jnp.int32),
        ],
    )(xt)
    combine_weight = jnp.transpose(cw_ect, (2, 0, 1))
    sec_mask = jnp.transpose(mask_ect, (2, 0, 1))
    return combine_weight, sec_mask


# speedup vs baseline: 1.0803x; 1.0803x over previous
import functools
import math

import jax
import jax.numpy as jnp
from jax import lax
from jax.experimental import pallas as pl
from jax.experimental.pallas import tpu as pltpu
from jax.experimental.pallas import tpu_sc as plsc

T, E = 2048, 64
CAP = 64
NS = 16
TPW = T // NS
NG = TPW // 16

_MESH = plsc.VectorSubcoreMesh(core_axis_name="c", subcore_axis_name="s")
_PARAMS = pltpu.CompilerParams(needs_layout_passes=False)


def _iota16():
    return lax.broadcasted_iota(jnp.int32, (16,), 0)


def _perm(v, idx):
    return lax.gather(
        v, idx[:, None],
        lax.GatherDimensionNumbers(
            offset_dims=(), collapsed_slice_dims=(0,), start_index_map=(0,)),
        (1,), mode=lax.GatherScatterMode.PROMISE_IN_BOUNDS)


def _group_rank(ev, iota, one, zeros):
    rl = jnp.zeros((16,), jnp.int32)
    fw = jnp.zeros((16,), jnp.int32)
    for s in range(1, 16):
        bk = jnp.maximum(iota - s, 0)
        fd = jnp.minimum(iota + s, 15)
        rl = rl + jnp.where((iota >= s) & (_perm(ev, bk) == ev), one, zeros)
        fw = fw + jnp.where((iota + s < 16) & (_perm(ev, fd) == ev), one, zeros)
    return rl, fw


def _sc_a_body(x_hbm, cnt1_hbm, cnt2_hbm, e1_hbm, e2_hbm, p1_hbm, p2_hbm,
               rl1_hbm, rl2_hbm, ls1_hbm, ls2_hbm,
               xbuf, e1buf, e2buf, p1buf, p2buf,
               rl1buf, rl2buf, ls1buf, ls2buf, cnt1, cnt2):
    cid = lax.axis_index("c")
    sid = lax.axis_index("s")

    @pl.when(cid == 0)
    def _body():
        base = sid * TPW
        pltpu.sync_copy(x_hbm.at[pl.ds(base, TPW)], xbuf)
        iota = _iota16()
        zeros = jnp.zeros((16,), jnp.int32)
        one = jnp.ones((16,), jnp.int32)
        for k in range(E // 16):
            cnt1[pl.ds(16 * k, 16)] = zeros
            cnt2[pl.ds(16 * k, 16)] = zeros

        def phase1(g, _):
            tokv = g * 16 + iota

            def col(e):
                return plsc.load_gather(
                    xbuf, [tokv, jnp.full((16,), e, jnp.int32)])

            mv = col(0)
            for e in range(1, E):
                mv = jnp.maximum(mv, col(e))
            zv = jnp.zeros((16,), jnp.float32)
            for e in range(E):
                zv = zv + jnp.exp(col(e) - mv)
            m1v = jnp.full((16,), -1.0, jnp.float32)
            m2v = jnp.full((16,), -1.0, jnp.float32)
            e1v = jnp.zeros((16,), jnp.int32)
            e2v = jnp.zeros((16,), jnp.int32)
            for e in range(E):
                pv = jnp.exp(col(e) - mv) / zv
                gt1 = pv > m1v
                gt2 = pv > m2v
                ec = jnp.full((16,), e, jnp.int32)
                m2v = jnp.where(gt1, m1v, jnp.where(gt2, pv, m2v))
                e2v = jnp.where(gt1, e1v, jnp.where(gt2, ec, e2v))
                m1v = jnp.where(gt1, pv, m1v)
                e1v = jnp.where(gt1, ec, e1v)

            sl = pl.ds(g * 16, 16)
            e1buf[sl] = e1v
            e2buf[sl] = e2v
            p1buf[sl] = m1v
            p2buf[sl] = m2v

            rl1, fw1 = _group_rank(e1v, iota, one, zeros)
            rl2, fw2 = _group_rank(e2v, iota, one, zeros)
            rl1buf[sl] = rl1
            rl2buf[sl] = rl2
            ls1buf[sl] = fw1
            ls2buf[sl] = fw2

            old1 = plsc.load_gather(cnt1, [e1v])
            plsc.store_scatter(cnt1, [e1v], old1 + rl1 + 1, mask=fw1 == 0)
            old2 = plsc.load_gather(cnt2, [e2v])
            plsc.store_scatter(cnt2, [e2v], old2 + rl2 + 1, mask=fw2 == 0)
            return 0

        lax.fori_loop(0, NG, phase1, 0)

        pltpu.sync_copy(cnt1, cnt1_hbm.at[sid])
        pltpu.sync_copy(cnt2, cnt2_hbm.at[sid])
        pltpu.sync_copy(e1buf, e1_hbm.at[sid])
        pltpu.sync_copy(e2buf, e2_hbm.at[sid])
        pltpu.sync_copy(p1buf, p1_hbm.at[sid])
        pltpu.sync_copy(p2buf, p2_hbm.at[sid])
        pltpu.sync_copy(rl1buf, rl1_hbm.at[sid])
        pltpu.sync_copy(rl2buf, rl2_hbm.at[sid])
        pltpu.sync_copy(ls1buf, ls1_hbm.at[sid])
        pltpu.sync_copy(ls2buf, ls2_hbm.at[sid])


_sc_a = pl.kernel(
    _sc_a_body,
    mesh=_MESH,
    compiler_params=_PARAMS,
    out_type=[
        jax.ShapeDtypeStruct((NS, E), jnp.int32),
        jax.ShapeDtypeStruct((NS, E), jnp.int32),
        jax.ShapeDtypeStruct((NS, TPW), jnp.int32),
        jax.ShapeDtypeStruct((NS, TPW), jnp.int32),
        jax.ShapeDtypeStruct((NS, TPW), jnp.float32),
        jax.ShapeDtypeStruct((NS, TPW), jnp.float32),
        jax.ShapeDtypeStruct((NS, TPW), jnp.int32),
        jax.ShapeDtypeStruct((NS, TPW), jnp.int32),
        jax.ShapeDtypeStruct((NS, TPW), jnp.int32),
        jax.ShapeDtypeStruct((NS, TPW), jnp.int32),
    ],
    scratch_types=[
        pltpu.VMEM((TPW, E), jnp.float32),
        pltpu.VMEM((TPW,), jnp.int32),
        pltpu.VMEM((TPW,), jnp.int32),
        pltpu.VMEM((TPW,), jnp.float32),
        pltpu.VMEM((TPW,), jnp.float32),
        pltpu.VMEM((TPW,), jnp.int32),
        pltpu.VMEM((TPW,), jnp.int32),
        pltpu.VMEM((TPW,), jnp.int32),
        pltpu.VMEM((TPW,), jnp.int32),
        pltpu.VMEM((E,), jnp.int32),
        pltpu.VMEM((E,), jnp.int32),
    ],
)


def _sc_b_body(cnt1_hbm, cnt2_hbm, e1_hbm, e2_hbm, p1_hbm, p2_hbm,
               rl1_hbm, rl2_hbm, ls1_hbm, ls2_hbm,
               w3_hbm, r3_hbm,
               allcnt1, allcnt2, e1buf, e2buf, p1buf, p2buf,
               rl1buf, rl2buf, ls1buf, ls2buf, cnt1, cnt2, wblk, rblk):
    cid = lax.axis_index("c")
    sid = lax.axis_index("s")

    @pl.when(cid == 0)
    def _body():
        pltpu.sync_copy(cnt1_hbm, allcnt1)
        pltpu.sync_copy(cnt2_hbm, allcnt2)
        pltpu.sync_copy(e1_hbm.at[sid], e1buf)
        pltpu.sync_copy(e2_hbm.at[sid], e2buf)
        pltpu.sync_copy(p1_hbm.at[sid], p1buf)
        pltpu.sync_copy(p2_hbm.at[sid], p2buf)
        pltpu.sync_copy(rl1_hbm.at[sid], rl1buf)
        pltpu.sync_copy(rl2_hbm.at[sid], rl2buf)
        pltpu.sync_copy(ls1_hbm.at[sid], ls1buf)
        pltpu.sync_copy(ls2_hbm.at[sid], ls2buf)
        iota = _iota16()
        zeros = jnp.zeros((16,), jnp.int32)
        zerosf = jnp.zeros((16,), jnp.float32)
        neg1 = jnp.full((16,), -1, jnp.int32)

        pre1 = [jnp.zeros((16,), jnp.int32) for _ in range(E // 16)]
        pre2 = [jnp.zeros((16,), jnp.int32) for _ in range(E // 16)]
        tot1 = [jnp.zeros((16,), jnp.int32) for _ in range(E // 16)]
        for w in range(NS):
            for k in range(E // 16):
                row1 = allcnt1[w, pl.ds(16 * k, 16)]
                row2 = allcnt2[w, pl.ds(16 * k, 16)]
                pre1[k] = pre1[k] + jnp.where(sid > w, row1, zeros)
                pre2[k] = pre2[k] + jnp.where(sid > w, row2, zeros)
                tot1[k] = tot1[k] + row1
        for k in range(E // 16):
            cnt1[pl.ds(16 * k, 16)] = pre1[k]
            cnt2[pl.ds(16 * k, 16)] = pre2[k] + tot1[k]
        for e in range(E):
            for k in range(TPW // 16):
                wblk[e, pl.ds(16 * k, 16)] = zerosf
                rblk[e, pl.ds(16 * k, 16)] = neg1

        def phase3(g, _):
            sl = pl.ds(g * 16, 16)
            tokl = g * 16 + iota
            e1v = e1buf[sl]
            e2v = e2buf[sl]
            rl1 = rl1buf[sl]
            rl2 = rl2buf[sl]
            lo1 = ls1buf[sl] == 0
            lo2 = ls2buf[sl] == 0
            old1 = plsc.load_gather(cnt1, [e1v])
            r1 = old1 + rl1
            plsc.store_scatter(cnt1, [e1v], r1 + 1, mask=lo1)
            old2 = plsc.load_gather(cnt2, [e2v])
            r2 = old2 + rl2
            plsc.store_scatter(cnt2, [e2v], r2 + 1, mask=lo2)
            k1 = r1 < CAP
            k2 = r2 < CAP
            plsc.store_scatter(wblk, [e1v, tokl], p1buf[sl], mask=k1)
            plsc.store_scatter(rblk, [e1v, tokl], r1, mask=k1)
            plsc.store_scatter(wblk, [e2v, tokl], p2buf[sl], mask=k2)
            plsc.store_scatter(rblk, [e2v, tokl], r2, mask=k2)
            return 0

        lax.fori_loop(0, NG, phase3, 0)

        pltpu.sync_copy(wblk, w3_hbm.at[sid])
        pltpu.sync_copy(rblk, r3_hbm.at[sid])


_sc_b = pl.kernel(
    _sc_b_body,
    mesh=_MESH,
    compiler_params=_PARAMS,
    out_type=[
        jax.ShapeDtypeStruct((NS, E, TPW), jnp.float32),
        jax.ShapeDtypeStruct((NS, E, TPW), jnp.int32),
    ],
    scratch_types=[
        pltpu.VMEM((NS, E), jnp.int32),
        pltpu.VMEM((NS, E), jnp.int32),
        pltpu.VMEM((TPW,), jnp.int32),
        pltpu.VMEM((TPW,), jnp.int32),
        pltpu.VMEM((TPW,), jnp.float32),
        pltpu.VMEM((TPW,), jnp.float32),
        pltpu.VMEM((TPW,), jnp.int32),
        pltpu.VMEM((TPW,), jnp.int32),
        pltpu.VMEM((TPW,), jnp.int32),
        pltpu.VMEM((TPW,), jnp.int32),
        pltpu.VMEM((E,), jnp.int32),
        pltpu.VMEM((E,), jnp.int32),
        pltpu.VMEM((E, TPW), jnp.float32),
        pltpu.VMEM((E, TPW), jnp.int32),
    ],
)


def _sc_meta(x):
    outs = _sc_a(x)
    return _sc_b(*outs)


def _fill_body(capacity, w_ref, r_ref, cw_ref, mask_ref):
    EB, t = w_ref.shape
    w = w_ref[...][:, None, :]
    r = r_ref[...][:, None, :]
    jc = jax.lax.broadcasted_iota(jnp.int32, (EB, capacity, t), 1)
    out = jnp.where(jc == r, w, 0.0)
    cw_ref[...] = out
    mask_ref[...] = out != 0.0


@jax.jit
def kernel(inputs):
    t, e = inputs.shape
    capacity = math.floor(2.0 * t / e)
    capacity += capacity % 2
    capacity = max(capacity, 4)

    w3, r3 = _sc_meta(inputs.astype(jnp.float32))
    W = jnp.transpose(w3, (1, 0, 2)).reshape(e, t)
    R = jnp.transpose(r3, (1, 0, 2)).reshape(e, t)

    EB = 8
    cw_ect, mask_ect = pl.pallas_call(
        functools.partial(_fill_body, capacity),
        grid=(e // EB,),
        in_specs=[
            pl.BlockSpec((EB, t), lambda i: (i, 0)),
            pl.BlockSpec((EB, t), lambda i: (i, 0)),
        ],
        out_specs=[
            pl.BlockSpec((EB, capacity, t), lambda i: (i, 0, 0)),
            pl.BlockSpec((EB, capacity, t), lambda i: (i, 0, 0)),
        ],
        out_shape=[
            jax.ShapeDtypeStruct((e, capacity, t), jnp.float32),
            jax.ShapeDtypeStruct((e, capacity, t), jnp.bool_),
        ],
    )(W, R)
    combine_weight = jnp.transpose(cw_ect, (2, 0, 1))
    sec_mask = jnp.transpose(mask_ect, (2, 0, 1))
    return combine_weight, sec_mask


# speedup vs baseline: 2.2388x; 2.0724x over previous
import functools
import math

import jax
import jax.numpy as jnp
from jax.experimental import pallas as pl
from jax.experimental.pallas import tpu as pltpu


def _router_body(capacity, xt_ref, cw_ref, mask_ref, w_ref, r_ref):
    i = pl.program_id(0)
    E, T = xt_ref.shape
    EB = cw_ref.shape[0]

    @pl.when(i == 0)
    def _compute_metadata():
        x = xt_ref[...]
        m = jnp.max(x, axis=0, keepdims=True)
        ex = jnp.exp(x - m)
        p = ex / jnp.sum(ex, axis=0, keepdims=True)

        eids = jax.lax.broadcasted_iota(jnp.int32, (E, T), 0)
        p1 = jnp.max(p, axis=0, keepdims=True)
        idx1 = jnp.min(jnp.where(p == p1, eids, E), axis=0, keepdims=True)
        m1 = eids == idx1
        pm = jnp.where(m1, -jnp.inf, p)
        p2 = jnp.max(pm, axis=0, keepdims=True)
        idx2 = jnp.min(jnp.where(pm == p2, eids, E), axis=0, keepdims=True)
        m2 = eids == idx2

        def _cumsum1(v):
            s = 1
            while s < v.shape[1]:
                z = jnp.zeros((v.shape[0], s), v.dtype)
                v = v + jnp.concatenate([z, v[:, :-s]], axis=1)
                s *= 2
            return v

        c1 = _cumsum1(m1.astype(jnp.int32))
        c2 = _cumsum1(m2.astype(jnp.int32))
        total1 = c1[:, T - 1:T]
        rank1 = jnp.sum(jnp.where(m1, c1, 0), axis=0, keepdims=True) - 1
        rank2 = jnp.sum(jnp.where(m2, c2 + total1, 0), axis=0, keepdims=True) - 1

        keep1 = m1 & (rank1 < capacity)
        keep2 = m2 & (rank2 < capacity)
        w_ref[...] = jnp.where(keep1 | keep2, p, 0.0)
        r_ref[...] = (jnp.where(keep1, rank1, -1)
                      + jnp.where(keep2, rank2 + 1, 0))

    w = w_ref[pl.ds(i * EB, EB), :][:, None, :]
    r = r_ref[pl.ds(i * EB, EB), :][:, None, :]
    jc = jax.lax.broadcasted_iota(jnp.int32, (EB, capacity, T), 1)
    out = jnp.where(jc == r, w, 0.0)
    cw_ref[...] = out
    mask_ref[...] = out != 0.0


@jax.jit
def kernel(inputs):
    T, E = inputs.shape
    capacity = math.floor(2.0 * T / E)
    capacity += capacity % 2
    capacity = max(capacity, 4)

    xt = jnp.swapaxes(inputs.astype(jnp.float32), 0, 1)
    EB = 8
    cw_ect, mask_ect = pl.pallas_call(
        functools.partial(_router_body, capacity),
        grid=(E // EB,),
        in_specs=[pl.BlockSpec((E, T), lambda i: (0, 0))],
        out_specs=[
            pl.BlockSpec((EB, capacity, T), lambda i: (i, 0, 0)),
            pl.BlockSpec((EB, capacity, T), lambda i: (i, 0, 0)),
        ],
        out_shape=[
            jax.ShapeDtypeStruct((E, capacity, T), jnp.float32),
            jax.ShapeDtypeStruct((E, capacity, T), jnp.bool_),
        ],
        scratch_shapes=[
            pltpu.VMEM((E, T), jnp.float32),
            pltpu.VMEM((E, T), jnp.int32),
        ],
    )(xt)
    combine_weight = jnp.transpose(cw_ect, (2, 0, 1))
    sec_mask = jnp.transpose(mask_ect, (2, 0, 1))
    return combine_weight, sec_mask
